# baseline (speedup 1.0000x reference)
import functools
import jax
import jax.numpy as jnp
from jax import lax
from jax.experimental import pallas as pl
from jax.experimental.pallas import tpu as pltpu
from jax.experimental.pallas import tpu_sc as plsc

_L = 16


def _make_sc_kernel(B, S, D):
    NC, NS = 2, 16
    NW = NC * NS
    SPW = S // NW
    CHUNK = 32
    NCHUNK = SPW // CHUNK
    VECS = CHUNK * D // _L
    mesh = plsc.VectorSubcoreMesh(core_axis_name="c", subcore_axis_name="s")

    @functools.partial(
        pl.kernel,
        mesh=mesh,
        out_type=jax.ShapeDtypeStruct((B, S, D), jnp.float32),
        scratch_types=[
            pltpu.VMEM((CHUNK, D), jnp.float32),
            pltpu.VMEM((CHUNK, D), jnp.float32),
        ],
    )
    def k(x_hbm, pos_hbm, out_hbm, xbuf, pbuf):
        wid = lax.axis_index("s") * NC + lax.axis_index("c")
        base = wid * SPW

        def chunk_body(c, carry):
            r0 = base + c * CHUNK
            pltpu.sync_copy(pos_hbm.at[pl.ds(r0, CHUNK)], pbuf)
            for b in range(B):
                pltpu.sync_copy(x_hbm.at[b, pl.ds(r0, CHUNK)], xbuf)

                def add_body(i, acc):
                    row = i // (D // _L)
                    col = (i % (D // _L)) * _L
                    xbuf[row, pl.ds(col, _L)] = (
                        xbuf[row, pl.ds(col, _L)] + pbuf[row, pl.ds(col, _L)]
                    )
                    return acc

                lax.fori_loop(0, VECS, add_body, 0)
                pltpu.sync_copy(xbuf, out_hbm.at[b, pl.ds(r0, CHUNK)])
            return carry

        lax.fori_loop(0, NCHUNK, chunk_body, 0)

    return k


def kernel(x, pos_emb):
    B, S, D = x.shape
    return _make_sc_kernel(B, S, D)(x, pos_emb)


# speedup vs baseline: 4.9571x; 4.9571x over previous
import jax
import jax.numpy as jnp
from jax.experimental import pallas as pl
from jax.experimental.pallas import tpu as pltpu


def _add_kernel(x_ref, p_ref, o_ref):
    o_ref[...] = x_ref[...] + p_ref[...]


def kernel(x, pos_emb):
    B, S, D = x.shape
    RBLK = 2048
    nseq = S // RBLK
    x2 = x.reshape(B * S, D)

    def xmap(i):
        return ((i % B) * nseq + i // B, 0)

    out = pl.pallas_call(
        _add_kernel,
        grid=(nseq * B,),
        in_specs=[
            pl.BlockSpec((RBLK, D), xmap),
            pl.BlockSpec((RBLK, D), lambda i: (i // B, 0)),
        ],
        out_specs=pl.BlockSpec((RBLK, D), xmap),
        out_shape=jax.ShapeDtypeStruct((B * S, D), x.dtype),
        compiler_params=pltpu.CompilerParams(
            dimension_semantics=("parallel",),
        ),
    )(x2, pos_emb)
    return out.reshape(B, S, D)
